# baseline (speedup 1.0000x reference)
import functools

import jax
import jax.numpy as jnp
from jax import lax
from jax.experimental import pallas as pl
from jax.experimental.pallas import tpu as pltpu
from jax.experimental.pallas import tpu_sc as plsc

_TOPK = 2048
_INV_T = 10.0
_H = 512
_W = 512
_B = 8
_NTILES = 32
_ROWS_PER_KP = 10
_CHUNK = 128
_M_IMG = 32768
_CAP = 16384 + 256
_SCHUNK = 8192

_SC_PARAMS = pltpu.CompilerParams(
    needs_layout_passes=False, use_tc_tiling_on_sc=False)


def _nms_body(s_ref, o_ref):
    x = s_ref[0, 0]
    neg = jnp.float32(-jnp.inf)

    def mp(a):
        h, w = a.shape
        nr = jnp.full((2, w), neg, jnp.float32)
        m = jnp.maximum(
            a,
            jnp.maximum(
                jnp.maximum(jnp.concatenate([a[1:], nr[:1]], 0),
                            jnp.concatenate([nr[:1], a[:-1]], 0)),
                jnp.maximum(jnp.concatenate([a[2:], nr], 0),
                            jnp.concatenate([nr, a[:-2]], 0))))
        nc = jnp.full((h, 2), neg, jnp.float32)
        m = jnp.maximum(
            m,
            jnp.maximum(
                jnp.maximum(jnp.concatenate([m[:, 1:], nc[:, :1]], 1),
                            jnp.concatenate([nc[:, :1], m[:, :-1]], 1)),
                jnp.maximum(jnp.concatenate([m[:, 2:], nc], 1),
                            jnp.concatenate([nc, m[:, :-2]], 1))))
        return m

    zeros = jnp.zeros_like(x)
    ones = jnp.ones_like(x)
    max_mask = x == mp(x)
    for _ in range(2):
        supp = mp(jnp.where(max_mask, ones, zeros)) > 0
        supp_scores = jnp.where(supp, zeros, x)
        new_max = supp_scores == mp(supp_scores)
        max_mask = max_mask | (new_max & (~supp))
    nms = jnp.where(max_mask, x, zeros)
    row = lax.broadcasted_iota(jnp.int32, (_H, _W), 0)
    col = lax.broadcasted_iota(jnp.int32, (_H, _W), 1)
    inb = (row >= 2) & (row < _H - 2) & (col >= 2) & (col < _W - 2)
    o_ref[0] = jnp.where(inb, nms, zeros)


@functools.lru_cache(maxsize=2)
def _nms_call(nimg):
    return pl.pallas_call(
        _nms_body,
        grid=(nimg,),
        in_specs=[pl.BlockSpec((1, 1, _H, _W), lambda i: (i, 0, 0, 0))],
        out_specs=pl.BlockSpec((1, _H, _W), lambda i: (i, 0, 0)),
        out_shape=jax.ShapeDtypeStruct((nimg, _H, _W), jnp.float32),
    )


def _make_compact_body(nimg):
    ipc = nimg // 2
    tpi = 16 // ipc
    strip = _H * _W // tpi

    def body(nms_hbm, vals_hbm, idxc_hbm,
             strm, vals, idxs, shared, stage, zv, sem):
        cid = lax.axis_index("c")
        sid = lax.axis_index("s")
        b = cid * ipc + sid // tpi
        q = sid % tpi
        sbase = b * (_H * _W) + q * strip
        lbase = q * strip
        lane = jax.lax.iota(jnp.int32, 16)

        def chunk(ci, off):
            pltpu.sync_copy(nms_hbm.at[pl.ds(sbase + ci * _SCHUNK, _SCHUNK)],
                            strm)

            def scan(i, off):
                v = strm[pl.ds(i * 16, 16)]
                m = (v > 0.0) & (off < _CAP - 272)
                iv = lbase + ci * _SCHUNK + i * 16 + lane
                plsc.store_compressed(vals.at[pl.ds(off, 16)], v, mask=m)
                plsc.store_compressed(idxs.at[pl.ds(off, 16)], iv, mask=m)
                cntv = plsc.all_reduce_population_count(m)
                return off + lax.squeeze(lax.slice(cntv, (0,), (1,)), (0,))

            return lax.fori_loop(0, _SCHUNK // 16, scan, off)

        cnt = lax.fori_loop(0, strip // _SCHUNK, chunk, jnp.int32(0))

        zero16 = jnp.zeros((16,), jnp.float32)
        for j in range(16):
            vals[pl.ds(cnt + j * 16, 16)] = zero16
            zv[pl.ds(j * 16, 16)] = zero16
        cnt_pad = jnp.bitwise_and(cnt + 255, -256)

        stage[pl.ds(0, 16)] = jnp.zeros((16,), jnp.int32) + cnt_pad
        pltpu.sync_copy(stage, shared.at[sid])
        plsc.subcore_barrier()
        off_out = jnp.int32(0)
        total = jnp.int32(0)
        for j in range(tpi):
            pltpu.sync_copy(shared.at[(sid // tpi) * tpi + j], stage)
            cj = jnp.max(stage[pl.ds(0, 16)])
            off_out = off_out + jnp.where(jnp.int32(j) < q, cj, 0)
            total = total + cj

        dst0 = pl.multiple_of(b * _M_IMG + off_out, 256)

        def cp(i, _):
            pltpu.sync_copy(vals.at[pl.ds(i * 256, 256)],
                            vals_hbm.at[pl.ds(dst0 + i * 256, 256)])
            pltpu.sync_copy(idxs.at[pl.ds(i * 256, 256)],
                            idxc_hbm.at[pl.ds(dst0 + i * 256, 256)])
            return 0

        nblk = lax.min(cnt_pad, jnp.maximum(_M_IMG - off_out, 0))
        lax.fori_loop(0, lax.shift_right_logical(nblk, 8), cp, 0)

        @pl.when(q == tpi - 1)
        def _():
            zstart = pl.multiple_of(b * _M_IMG + total, 256)

            def zf(i, _):
                pltpu.sync_copy(zv, vals_hbm.at[pl.ds(zstart + i * 256, 256)])
                return 0

            lax.fori_loop(0, lax.shift_right_logical(_M_IMG - total, 8),
                          zf, 0)

    return body


@functools.lru_cache(maxsize=2)
def _compact_call(nimg):
    return pl.kernel(
        _make_compact_body(nimg),
        out_type=[
            jax.ShapeDtypeStruct((nimg * _M_IMG,), jnp.float32),
            jax.ShapeDtypeStruct((nimg * _M_IMG,), jnp.int32),
        ],
        mesh=plsc.VectorSubcoreMesh(core_axis_name="c", subcore_axis_name="s"),
        compiler_params=_SC_PARAMS,
        scratch_types=[
            pltpu.VMEM((_SCHUNK,), jnp.float32),
            pltpu.VMEM((_CAP,), jnp.float32),
            pltpu.VMEM((_CAP,), jnp.int32),
            pltpu.VMEM_SHARED((16, 16), jnp.int32),
            pltpu.VMEM((16,), jnp.int32),
            pltpu.VMEM((256,), jnp.float32),
            pltpu.SemaphoreType.DMA,
        ],
    )


def _make_refine_body(nimg):
    kpt = nimg * _TOPK // _NTILES
    tpi = _NTILES // nimg
    nrows = kpt * _ROWS_PER_KP
    nchunks = nrows // _CHUNK
    grp = 5 if nchunks % 5 == 0 else 8

    def body(s_hbm, pos_hbm, idxc_hbm, kx_hbm, ky_hbm, disp_hbm, ks_hbm,
             posv, icv, idxv, ribuf, patch, kxb, kyb, dispb, ksb, sem):
        cid = lax.axis_index("c")
        sid = lax.axis_index("s")
        wid = sid * 2 + cid
        bimg = wid // tpi
        kbase = wid * kpt

        pltpu.sync_copy(pos_hbm.at[pl.ds(kbase, kpt)], posv)
        pltpu.sync_copy(idxc_hbm.at[pl.ds(bimg * _M_IMG, _M_IMG)], icv)

        lane = jax.lax.iota(jnp.int32, 16)

        def trans(g, _):
            p16 = posv[pl.ds(g * 16, 16)]
            idxv[pl.ds(g * 16, 16)] = plsc.load_gather(icv, [p16])
            return 0

        lax.fori_loop(0, kpt // 16, trans, 0)

        def build(g, _):
            i16 = idxv[pl.ds(g * 16, 16)]
            y = lax.shift_right_logical(i16, 9)
            xk = jnp.bitwise_and(i16, _W - 1)
            c0 = lax.shift_right_logical(xk - 2, 4)
            base = bimg * (_H * _W // 16) + (y - 2) * (_W // 16) + c0
            for dy in range(5):
                for cc in range(2):
                    off = dy * 2 + cc
                    ribuf[pl.ds(off * kpt + g * 16, 16)] = (
                        base + dy * (_W // 16) + cc)
            return 0

        lax.fori_loop(0, kpt // 16, build, 0)

        def dma(gi, _):
            base = gi * grp
            for j in range(grp):
                pltpu.async_copy(
                    s_hbm.at[ribuf.at[pl.ds((base + j) * _CHUNK, _CHUNK)]],
                    patch.at[pl.ds((base + j) * _CHUNK, _CHUNK)], sem)
            for j in range(grp):
                pltpu.make_async_copy(
                    s_hbm.at[ribuf.at[pl.ds((base + j) * _CHUNK, _CHUNK)]],
                    patch.at[pl.ds((base + j) * _CHUNK, _CHUNK)], sem).wait()
            return 0

        lax.fori_loop(0, nchunks // grp, dma, 0)

        def comp(g, _):
            i16 = idxv[pl.ds(g * 16, 16)]
            y = lax.shift_right_logical(i16, 9)
            xk = jnp.bitwise_and(i16, _W - 1)
            xoff = jnp.bitwise_and(xk - 2, 15)
            kp = g * 16 + lane

            def tap(dy_row, ca):
                cc = lax.shift_right_logical(ca, 4)
                col = jnp.bitwise_and(ca, 15)
                return plsc.load_gather(
                    patch, [(dy_row * 2 + cc) * kpt + kp, col])

            s0 = jnp.zeros((16,), jnp.float32)
            sx = jnp.zeros((16,), jnp.float32)
            sy = jnp.zeros((16,), jnp.float32)
            s2 = jnp.zeros((16,), jnp.float32)
            for dy in range(5):
                for dx in range(5):
                    v = tap(dy, xoff + dx)
                    e = jnp.exp(v * _INV_T)
                    s0 = s0 + e
                    wx = float(dx - 2)
                    wy = float(dy - 2)
                    if wx:
                        sx = sx + e * wx
                    if wy:
                        sy = sy + e * wy
                    w2 = wx * wx + wy * wy
                    if w2:
                        s2 = s2 + e * w2
            rx = sx / s0
            ry = sy / s0
            disp = (s2 / s0 - rx * rx - ry * ry) * 0.25
            xf = xk.astype(jnp.float32)
            yf = y.astype(jnp.float32)
            kx = (xf + rx) / (_W - 1) * 2.0 - 1.0
            ky = (yf + ry) / (_H - 1) * 2.0 - 1.0
            px = (kx + 1.0) / 2.0 * (_W - 1)
            py = (ky + 1.0) / 2.0 * (_H - 1)
            x0i = px.astype(jnp.int32)
            y0i = py.astype(jnp.int32)
            wxf = px - x0i.astype(jnp.float32)
            wyf = py - y0i.astype(jnp.float32)
            dyb = y0i - y + 2
            cab = xoff + (x0i - xk + 2)
            v00 = tap(dyb, cab)
            v01 = tap(dyb, cab + 1)
            v10 = tap(dyb + 1, cab)
            v11 = tap(dyb + 1, cab + 1)
            ks = (v00 * (1.0 - wxf) * (1.0 - wyf) + v01 * wxf * (1.0 - wyf)
                  + v10 * (1.0 - wxf) * wyf + v11 * wxf * wyf)
            kxb[pl.ds(g * 16, 16)] = kx
            kyb[pl.ds(g * 16, 16)] = ky
            dispb[pl.ds(g * 16, 16)] = disp
            ksb[pl.ds(g * 16, 16)] = ks
            return 0

        lax.fori_loop(0, kpt // 16, comp, 0)

        pltpu.sync_copy(kxb, kx_hbm.at[pl.ds(kbase, kpt)])
        pltpu.sync_copy(kyb, ky_hbm.at[pl.ds(kbase, kpt)])
        pltpu.sync_copy(dispb, disp_hbm.at[pl.ds(kbase, kpt)])
        pltpu.sync_copy(ksb, ks_hbm.at[pl.ds(kbase, kpt)])

    return body


@functools.lru_cache(maxsize=2)
def _refine_call(nimg):
    kpt = nimg * _TOPK // _NTILES
    nrows = kpt * _ROWS_PER_KP
    return pl.kernel(
        _make_refine_body(nimg),
        out_type=[
            jax.ShapeDtypeStruct((nimg * _TOPK,), jnp.float32),
            jax.ShapeDtypeStruct((nimg * _TOPK,), jnp.float32),
            jax.ShapeDtypeStruct((nimg * _TOPK,), jnp.float32),
            jax.ShapeDtypeStruct((nimg * _TOPK,), jnp.float32),
        ],
        mesh=plsc.VectorSubcoreMesh(core_axis_name="c", subcore_axis_name="s"),
        compiler_params=_SC_PARAMS,
        scratch_types=[
            pltpu.VMEM((kpt,), jnp.int32),
            pltpu.VMEM((_M_IMG,), jnp.int32),
            pltpu.VMEM((kpt,), jnp.int32),
            pltpu.VMEM((nrows,), jnp.int32),
            pltpu.VMEM((nrows, 16), jnp.float32),
            pltpu.VMEM((kpt,), jnp.float32),
            pltpu.VMEM((kpt,), jnp.float32),
            pltpu.VMEM((kpt,), jnp.float32),
            pltpu.VMEM((kpt,), jnp.float32),
            pltpu.SemaphoreType.DMA,
        ],
    )


def _run_half(s_half, nimg):
    nms = _nms_call(nimg)(s_half)
    vals_c, idx_c = _compact_call(nimg)(nms.reshape(-1))
    _, pos = lax.top_k(vals_c.reshape(nimg, _M_IMG), _TOPK)
    s2d = s_half.reshape(nimg * _H * _W // 16, 16)
    return _refine_call(nimg)(s2d, pos.reshape(-1), idx_c)


def kernel(scores_map):
    b, c, h, w = scores_map.shape
    half = b // 2
    parts = [_run_half(lax.slice_in_dim(scores_map, i * half, (i + 1) * half,
                                        axis=0), half)
             for i in range(2)]
    kx, ky, disp, ks = (jnp.concatenate([p[i] for p in parts])
                        for i in range(4))
    kxy = jnp.stack([kx, ky], axis=-1)
    return (kxy.reshape(b, _TOPK, 2), disp.reshape(b, _TOPK),
            ks.reshape(b, _TOPK))


# speedup vs baseline: 1.3034x; 1.3034x over previous
import functools

import jax
import jax.numpy as jnp
from jax import lax
from jax.experimental import pallas as pl
from jax.experimental.pallas import tpu as pltpu
from jax.experimental.pallas import tpu_sc as plsc

_R = 2
_K = 5
_TOPK = 2048
_INV_T = 10.0
_H = 512
_W = 512
_B = 8
_NTILES = 32
_KP_PER_TILE = (_B * _TOPK) // _NTILES
_ROWS_PER_KP = 10
_NROWS = _KP_PER_TILE * _ROWS_PER_KP
_CHUNK = 128
_NCHUNKS = _NROWS // _CHUNK


def _nms_body(s_ref, o_ref):
    x = s_ref[0, 0]
    neg = jnp.float32(-jnp.inf)

    def mp(a):
        h, w = a.shape
        nr = jnp.full((2, w), neg, jnp.float32)
        m = jnp.maximum(
            a,
            jnp.maximum(
                jnp.maximum(jnp.concatenate([a[1:], nr[:1]], 0),
                            jnp.concatenate([nr[:1], a[:-1]], 0)),
                jnp.maximum(jnp.concatenate([a[2:], nr], 0),
                            jnp.concatenate([nr, a[:-2]], 0))))
        nc = jnp.full((h, 2), neg, jnp.float32)
        m = jnp.maximum(
            m,
            jnp.maximum(
                jnp.maximum(jnp.concatenate([m[:, 1:], nc[:, :1]], 1),
                            jnp.concatenate([nc[:, :1], m[:, :-1]], 1)),
                jnp.maximum(jnp.concatenate([m[:, 2:], nc], 1),
                            jnp.concatenate([nc, m[:, :-2]], 1))))
        return m

    zeros = jnp.zeros_like(x)
    ones = jnp.ones_like(x)
    max_mask = x == mp(x)
    for _ in range(2):
        supp = mp(jnp.where(max_mask, ones, zeros)) > 0
        supp_scores = jnp.where(supp, zeros, x)
        new_max = supp_scores == mp(supp_scores)
        max_mask = max_mask | (new_max & (~supp))
    nms = jnp.where(max_mask, x, zeros)
    row = lax.broadcasted_iota(jnp.int32, (_H, _W), 0)
    col = lax.broadcasted_iota(jnp.int32, (_H, _W), 1)
    inb = (row >= _R) & (row < _H - _R) & (col >= _R) & (col < _W - _R)
    o_ref[0] = jnp.where(inb, nms, zeros)


_nms_call = pl.pallas_call(
    _nms_body,
    grid=(_B,),
    in_specs=[pl.BlockSpec((1, 1, _H, _W), lambda i: (i, 0, 0, 0))],
    out_specs=pl.BlockSpec((1, _H, _W), lambda i: (i, 0, 0)),
    out_shape=jax.ShapeDtypeStruct((_B, _H, _W), jnp.float32),
)


_M_IMG = 32768
_STRIP = _H * _W // 4
_CAP = 16384 + 256
_SCHUNK = 8192


def _compact_body(nms_hbm, vals_hbm, idxc_hbm,
                  strm, vals, idxs, shared, stage, zv, sem):
    cid = lax.axis_index("c")
    sid = lax.axis_index("s")
    b = cid * 4 + sid // 4
    q = sid % 4
    sbase = b * (_H * _W) + q * _STRIP
    lbase = q * _STRIP
    lane = jax.lax.iota(jnp.int32, 16)

    def chunk(ci, off):
        pltpu.sync_copy(nms_hbm.at[pl.ds(sbase + ci * _SCHUNK, _SCHUNK)],
                        strm)

        def scan(i, off):
            v = strm[pl.ds(i * 16, 16)]
            m = (v > 0.0) & (off < _CAP - 272)
            iv = lbase + ci * _SCHUNK + i * 16 + lane
            plsc.store_compressed(vals.at[pl.ds(off, 16)], v, mask=m)
            plsc.store_compressed(idxs.at[pl.ds(off, 16)], iv, mask=m)
            cntv = plsc.all_reduce_population_count(m)
            return off + lax.squeeze(lax.slice(cntv, (0,), (1,)), (0,))

        return lax.fori_loop(0, _SCHUNK // 16, scan, off)

    cnt = lax.fori_loop(0, _STRIP // _SCHUNK, chunk, jnp.int32(0))

    zero16 = jnp.zeros((16,), jnp.float32)
    for j in range(16):
        vals[pl.ds(cnt + j * 16, 16)] = zero16
        zv[pl.ds(j * 16, 16)] = zero16
    cnt_pad = jnp.bitwise_and(cnt + 255, -256)

    stage[pl.ds(0, 16)] = jnp.zeros((16,), jnp.int32) + cnt_pad
    pltpu.sync_copy(stage, shared.at[sid])
    plsc.subcore_barrier()
    off_out = jnp.int32(0)
    total = jnp.int32(0)
    for j in range(4):
        pltpu.sync_copy(shared.at[(sid // 4) * 4 + j], stage)
        cj = jnp.max(stage[pl.ds(0, 16)])
        off_out = off_out + jnp.where(jnp.int32(j) < q, cj, 0)
        total = total + cj

    dst0 = pl.multiple_of(b * _M_IMG + off_out, 256)

    def cp(i, _):
        pltpu.sync_copy(vals.at[pl.ds(i * 256, 256)],
                        vals_hbm.at[pl.ds(dst0 + i * 256, 256)])
        pltpu.sync_copy(idxs.at[pl.ds(i * 256, 256)],
                        idxc_hbm.at[pl.ds(dst0 + i * 256, 256)])
        return 0

    nblk = lax.min(cnt_pad, jnp.maximum(_M_IMG - off_out, 0))
    lax.fori_loop(0, lax.shift_right_logical(nblk, 8), cp, 0)

    @pl.when(q == 3)
    def _():
        zstart = pl.multiple_of(b * _M_IMG + total, 256)

        def zf(i, _):
            pltpu.sync_copy(zv, vals_hbm.at[pl.ds(zstart + i * 256, 256)])
            return 0

        lax.fori_loop(0, lax.shift_right_logical(_M_IMG - total, 8), zf, 0)


@functools.lru_cache(maxsize=1)
def _compact_call():
    return pl.kernel(
        _compact_body,
        out_type=[
            jax.ShapeDtypeStruct((_B * _M_IMG,), jnp.float32),
            jax.ShapeDtypeStruct((_B * _M_IMG,), jnp.int32),
        ],
        mesh=plsc.VectorSubcoreMesh(core_axis_name="c", subcore_axis_name="s"),
        compiler_params=pltpu.CompilerParams(
            needs_layout_passes=False, use_tc_tiling_on_sc=False),
        scratch_types=[
            pltpu.VMEM((_SCHUNK,), jnp.float32),
            pltpu.VMEM((_CAP,), jnp.float32),
            pltpu.VMEM((_CAP,), jnp.int32),
            pltpu.VMEM_SHARED((16, 16), jnp.int32),
            pltpu.VMEM((16,), jnp.int32),
            pltpu.VMEM((256,), jnp.float32),
            pltpu.SemaphoreType.DMA,
        ],
    )


def _refine_body(s_hbm, pos_hbm, idxc_hbm, kx_hbm, ky_hbm, disp_hbm, ks_hbm,
                 posv, icv, idxv, ribuf, patch, kxb, kyb, dispb, ksb, sem):
    cid = lax.axis_index("c")
    sid = lax.axis_index("s")
    wid = sid * 2 + cid
    bimg = wid // 4
    kbase = wid * _KP_PER_TILE

    pltpu.sync_copy(pos_hbm.at[pl.ds(kbase, _KP_PER_TILE)], posv)
    pltpu.sync_copy(idxc_hbm.at[pl.ds(bimg * _M_IMG, _M_IMG)], icv)

    lane = jax.lax.iota(jnp.int32, 16)

    def trans(g, _):
        p16 = posv[pl.ds(g * 16, 16)]
        idxv[pl.ds(g * 16, 16)] = plsc.load_gather(icv, [p16])
        return 0

    lax.fori_loop(0, _KP_PER_TILE // 16, trans, 0)

    def build(g, _):
        i16 = idxv[pl.ds(g * 16, 16)]
        y = lax.shift_right_logical(i16, 9)
        xk = jnp.bitwise_and(i16, _W - 1)
        c0 = lax.shift_right_logical(xk - 2, 4)
        base = bimg * (_H * _W // 16) + (y - 2) * (_W // 16) + c0
        for dy in range(5):
            for cc in range(2):
                off = dy * 2 + cc
                ribuf[pl.ds(off * _KP_PER_TILE + g * 16, 16)] = (
                    base + dy * (_W // 16) + cc)
        return 0

    lax.fori_loop(0, _KP_PER_TILE // 16, build, 0)

    def dma(gi, _):
        base = gi * 8
        for j in range(8):
            pltpu.async_copy(
                s_hbm.at[ribuf.at[pl.ds((base + j) * _CHUNK, _CHUNK)]],
                patch.at[pl.ds((base + j) * _CHUNK, _CHUNK)], sem)
        for j in range(8):
            pltpu.make_async_copy(
                s_hbm.at[ribuf.at[pl.ds((base + j) * _CHUNK, _CHUNK)]],
                patch.at[pl.ds((base + j) * _CHUNK, _CHUNK)], sem).wait()
        return 0

    lax.fori_loop(0, _NCHUNKS // 8, dma, 0)

    def comp(g, _):
        i16 = idxv[pl.ds(g * 16, 16)]
        y = lax.shift_right_logical(i16, 9)
        xk = jnp.bitwise_and(i16, _W - 1)
        xoff = jnp.bitwise_and(xk - 2, 15)
        kp = g * 16 + lane

        def tap(dy_row, ca):
            cc = lax.shift_right_logical(ca, 4)
            col = jnp.bitwise_and(ca, 15)
            return plsc.load_gather(
                patch, [(dy_row * 2 + cc) * _KP_PER_TILE + kp, col])

        s0 = jnp.zeros((16,), jnp.float32)
        sx = jnp.zeros((16,), jnp.float32)
        sy = jnp.zeros((16,), jnp.float32)
        s2 = jnp.zeros((16,), jnp.float32)
        for dy in range(5):
            for dx in range(5):
                v = tap(dy, xoff + dx)
                e = jnp.exp(v * _INV_T)
                s0 = s0 + e
                wx = float(dx - 2)
                wy = float(dy - 2)
                if wx:
                    sx = sx + e * wx
                if wy:
                    sy = sy + e * wy
                w2 = wx * wx + wy * wy
                if w2:
                    s2 = s2 + e * w2
        rx = sx / s0
        ry = sy / s0
        disp = (s2 / s0 - rx * rx - ry * ry) * 0.25
        xf = xk.astype(jnp.float32)
        yf = y.astype(jnp.float32)
        kx = (xf + rx) / (_W - 1) * 2.0 - 1.0
        ky = (yf + ry) / (_H - 1) * 2.0 - 1.0
        px = (kx + 1.0) / 2.0 * (_W - 1)
        py = (ky + 1.0) / 2.0 * (_H - 1)
        x0i = px.astype(jnp.int32)
        y0i = py.astype(jnp.int32)
        wxf = px - x0i.astype(jnp.float32)
        wyf = py - y0i.astype(jnp.float32)
        dyb = y0i - y + 2
        cab = xoff + (x0i - xk + 2)
        v00 = tap(dyb, cab)
        v01 = tap(dyb, cab + 1)
        v10 = tap(dyb + 1, cab)
        v11 = tap(dyb + 1, cab + 1)
        ks = (v00 * (1.0 - wxf) * (1.0 - wyf) + v01 * wxf * (1.0 - wyf)
              + v10 * (1.0 - wxf) * wyf + v11 * wxf * wyf)
        kxb[pl.ds(g * 16, 16)] = kx
        kyb[pl.ds(g * 16, 16)] = ky
        dispb[pl.ds(g * 16, 16)] = disp
        ksb[pl.ds(g * 16, 16)] = ks
        return 0

    lax.fori_loop(0, _KP_PER_TILE // 16, comp, 0)

    pltpu.sync_copy(kxb, kx_hbm.at[pl.ds(kbase, _KP_PER_TILE)])
    pltpu.sync_copy(kyb, ky_hbm.at[pl.ds(kbase, _KP_PER_TILE)])
    pltpu.sync_copy(dispb, disp_hbm.at[pl.ds(kbase, _KP_PER_TILE)])
    pltpu.sync_copy(ksb, ks_hbm.at[pl.ds(kbase, _KP_PER_TILE)])


@functools.lru_cache(maxsize=1)
def _refine_call():
    return pl.kernel(
        _refine_body,
        out_type=[
            jax.ShapeDtypeStruct((_B * _TOPK,), jnp.float32),
            jax.ShapeDtypeStruct((_B * _TOPK,), jnp.float32),
            jax.ShapeDtypeStruct((_B * _TOPK,), jnp.float32),
            jax.ShapeDtypeStruct((_B * _TOPK,), jnp.float32),
        ],
        mesh=plsc.VectorSubcoreMesh(core_axis_name="c", subcore_axis_name="s"),
        compiler_params=pltpu.CompilerParams(
            needs_layout_passes=False, use_tc_tiling_on_sc=False),
        scratch_types=[
            pltpu.VMEM((_KP_PER_TILE,), jnp.int32),
            pltpu.VMEM((_M_IMG,), jnp.int32),
            pltpu.VMEM((_KP_PER_TILE,), jnp.int32),
            pltpu.VMEM((_NROWS,), jnp.int32),
            pltpu.VMEM((_NROWS, 16), jnp.float32),
            pltpu.VMEM((_KP_PER_TILE,), jnp.float32),
            pltpu.VMEM((_KP_PER_TILE,), jnp.float32),
            pltpu.VMEM((_KP_PER_TILE,), jnp.float32),
            pltpu.VMEM((_KP_PER_TILE,), jnp.float32),
            pltpu.SemaphoreType.DMA,
        ],
    )


def kernel(scores_map):
    b, c, h, w = scores_map.shape
    nms = _nms_call(scores_map)
    vals_c, idx_c = _compact_call()(nms.reshape(-1))
    _, pos = lax.top_k(vals_c.reshape(b, _M_IMG), _TOPK)
    vals_c, idx_c = _compact_call()(nms.reshape(-1))
    _, pos = lax.top_k(vals_c.reshape(b, _M_IMG), _TOPK)
    s2d = scores_map.reshape(b * h * w // 16, 16)
    kx, ky, disp, ks = _refine_call()(s2d, pos.reshape(-1), idx_c)
    kxy = jnp.stack([kx, ky], axis=-1)
    return (kxy.reshape(b, _TOPK, 2), disp.reshape(b, _TOPK),
            ks.reshape(b, _TOPK))
